# baseline (speedup 1.0000x reference)
import functools

import jax
import jax.numpy as jnp
from jax import lax
from jax.experimental import pallas as pl
from jax.experimental.pallas import tpu as pltpu
from jax.experimental.pallas import tpu_sc as plsc

S_ = 2048
ROWS_ = 12 * 2048
NW_ = 32
RPW_ = ROWS_ // NW_
RCH_ = 16
NCH_ = RPW_ // RCH_


def _sc_body(x_hbm, cv_hbm, out_hbm, buf, cvv, sem):
    wid = lax.axis_index("s") * 2 + lax.axis_index("c")
    pltpu.sync_copy(cv_hbm, cvv)
    cv = cvv[...][0]
    shift = cv * 2048.0 - 1023.0

    row0 = wid * RPW_

    def chunk(ch, carry):
        base = row0 + ch * RCH_
        pltpu.sync_copy(x_hbm.at[pl.ds(base, RCH_), :], buf)
        for r in range(RCH_):
            rm = (base + r) & (S_ - 1)
            i = jnp.minimum(rm, S_ - 1 - rm)
            left = i + jnp.where(rm < S_ // 2, 0, 1)
            right = S_ - 1 - i
            odm = jnp.clip(i.astype(jnp.float32) * 0.03125 + shift * 0.03125
                           + 1.0, 0.0, 1.0)
            odm_v = jnp.full((16,), odm, dtype=jnp.float32)
            one_v = jnp.full((16,), 1.0, dtype=jnp.float32)
            left_v = jnp.full((16,), left, dtype=jnp.int32)
            right_v = jnp.full((16,), right, dtype=jnp.int32)
            ci = lax.iota(jnp.int32, 16)

            def vec(j, c2):
                c = ci + j * 16
                cond = (c >= left_v) & (c <= right_v)
                m = jnp.where(cond, odm_v, one_v)
                buf[r, pl.ds(j * 16, 16)] = buf[r, pl.ds(j * 16, 16)] * m
                return c2

            lax.fori_loop(0, S_ // 16, vec, 0)
        pltpu.sync_copy(buf, out_hbm.at[pl.ds(base, RCH_), :])
        return carry

    lax.fori_loop(0, NCH_, chunk, 0)


@jax.jit
def kernel(x, current_val):
    b, h, s, _ = x.shape
    x2 = x.reshape(ROWS_, S_)
    cv16 = jnp.broadcast_to(current_val, (16,))
    mesh = plsc.VectorSubcoreMesh(core_axis_name="c", subcore_axis_name="s")
    run = functools.partial(
        pl.kernel,
        mesh=mesh,
        out_type=jax.ShapeDtypeStruct((ROWS_, S_), jnp.float32),
        scratch_types=[
            pltpu.VMEM((RCH_, S_), jnp.float32),
            pltpu.VMEM((16,), jnp.float32),
            pltpu.SemaphoreType.DMA,
        ],
    )(_sc_body)
    out = run(x2, cv16)
    return out.reshape(b, h, s, s)


# speedup vs baseline: 3.2483x; 3.2483x over previous
import functools

import jax
import jax.numpy as jnp
from jax import lax
from jax.experimental import pallas as pl
from jax.experimental.pallas import tpu as pltpu
from jax.experimental.pallas import tpu_sc as plsc

S_ = 2048
ROWS_ = 12 * 2048
NW_ = 32
RPW_ = ROWS_ // NW_
RCH_ = 16
NCH_ = RPW_ // RCH_


def _sc_body(x_hbm, cv_hbm, out_hbm, buf, cvv, sem_in, sem_out):
    wid = lax.axis_index("s") * 2 + lax.axis_index("c")
    pltpu.sync_copy(cv_hbm, cvv)
    cv = cvv[...][0]
    shift = (cv * 2048.0 - 1023.0) * 0.03125 + 1.0

    row0 = wid * RPW_
    ci = lax.iota(jnp.int32, 16)

    def in_slice(ch):
        return x_hbm.at[pl.ds(row0 + ch * RCH_, RCH_), :]

    def out_slice(ch):
        return out_hbm.at[pl.ds(row0 + ch * RCH_, RCH_), :]

    def buf_slice(slot):
        return buf.at[pl.ds(slot * RCH_, RCH_), :]

    pltpu.async_copy(in_slice(0), buf_slice(0), sem_in)

    def chunk(ch, carry):
        slot = lax.rem(ch, 2)
        pltpu.make_async_copy(in_slice(ch), buf_slice(slot), sem_in).wait()

        @pl.when(ch + 1 < NCH_)
        def _():
            @pl.when(ch >= 1)
            def _():
                pltpu.make_async_copy(
                    buf_slice(1 - slot), out_slice(ch - 1), sem_out).wait()
            pltpu.async_copy(in_slice(ch + 1), buf_slice(1 - slot), sem_in)

        base = row0 + ch * RCH_
        for r in range(RCH_):
            rm = (base + r) & (S_ - 1)
            i = jnp.minimum(rm, S_ - 1 - rm)
            left = i + jnp.where(rm < S_ // 2, 0, 1)
            right = S_ - 1 - i
            odm = jnp.clip(i.astype(jnp.float32) * 0.03125 + shift, 0.0, 1.0)
            odm_v = jnp.full((16,), odm, dtype=jnp.float32)
            one_v = jnp.full((16,), 1.0, dtype=jnp.float32)
            left_v = jnp.full((16,), left, dtype=jnp.int32)
            right_v = jnp.full((16,), right, dtype=jnp.int32)
            roff = slot * RCH_ + r

            def vec(j, c2, roff=roff, left_v=left_v, right_v=right_v,
                    odm_v=odm_v, one_v=one_v):
                c = ci + j * 16
                cond = (c >= left_v) & (c <= right_v)
                m = jnp.where(cond, odm_v, one_v)
                buf[roff, pl.ds(j * 16, 16)] = buf[roff, pl.ds(j * 16, 16)] * m
                return c2

            lax.fori_loop(0, S_ // 16, vec, 0, unroll=8)

        pltpu.async_copy(buf_slice(slot), out_slice(ch), sem_out)
        return carry

    lax.fori_loop(0, NCH_, chunk, 0)
    last = NCH_ - 1
    pltpu.make_async_copy(
        buf_slice(lax.rem(last - 1, 2)), out_slice(last - 1), sem_out).wait()
    pltpu.make_async_copy(
        buf_slice(lax.rem(last, 2)), out_slice(last), sem_out).wait()


@jax.jit
def kernel(x, current_val):
    b, h, s, _ = x.shape
    x2 = x.reshape(ROWS_, S_)
    cv16 = jnp.broadcast_to(current_val, (16,))
    mesh = plsc.VectorSubcoreMesh(core_axis_name="c", subcore_axis_name="s")
    run = functools.partial(
        pl.kernel,
        mesh=mesh,
        out_type=jax.ShapeDtypeStruct((ROWS_, S_), jnp.float32),
        scratch_types=[
            pltpu.VMEM((2 * RCH_, S_), jnp.float32),
            pltpu.VMEM((16,), jnp.float32),
            pltpu.SemaphoreType.DMA,
            pltpu.SemaphoreType.DMA,
        ],
    )(_sc_body)
    out = run(x2, cv16)
    return out.reshape(b, h, s, s)


# speedup vs baseline: 5.0810x; 1.5642x over previous
import functools

import jax
import jax.numpy as jnp
from jax.experimental import pallas as pl
from jax.experimental.pallas import tpu as pltpu

MAX_SIZE_ = 2048
RAMP_ = 32.0


def _mask_mul_kernel(cv_ref, x_ref, o_ref, *, block_rows, s):
    cv = cv_ref[0]
    r = jax.lax.broadcasted_iota(jnp.int32, (block_rows, s), 0)
    r = (r + pl.program_id(0) * block_rows) & (s - 1)
    c = jax.lax.broadcasted_iota(jnp.int32, (block_rows, s), 1)
    i = jnp.minimum(r, s - 1 - r)
    left = i + jnp.where(r < s // 2, 0, 1)
    cond = (c >= left) & (c <= s - 1 - i)
    odm = (i.astype(jnp.float32) - (s // 2 - 1) + cv * MAX_SIZE_) / RAMP_ + 1.0
    odm = jnp.clip(odm, 0.0, 1.0)
    mask = jnp.where(cond, odm, 1.0)
    o_ref[...] = x_ref[...] * mask


@jax.jit
def kernel(x, current_val):
    b, h, s, _ = x.shape
    rows = b * h * s
    block_rows = 1536
    x2 = x.reshape(rows, s)
    grid = (rows // block_rows,)
    out = pl.pallas_call(
        functools.partial(_mask_mul_kernel, block_rows=block_rows, s=s),
        grid=grid,
        in_specs=[
            pl.BlockSpec(memory_space=pltpu.SMEM),
            pl.BlockSpec((block_rows, s), lambda n: (n, 0)),
        ],
        out_specs=pl.BlockSpec((block_rows, s), lambda n: (n, 0)),
        out_shape=jax.ShapeDtypeStruct((rows, s), x.dtype),
        compiler_params=pltpu.CompilerParams(
            dimension_semantics=("parallel",),
        ),
    )(current_val, x2)
    return out.reshape(b, h, s, s)
